# baseline (speedup 1.0000x reference)
import functools

import jax
import jax.numpy as jnp
from jax import lax
from jax.experimental import pallas as pl
from jax.experimental.pallas import tpu as pltpu
from jax.experimental.pallas import tpu_sc as plsc

_EMBED = 32
_L = 16
_NC = 2
_NS = 16
_NW = _NC * _NS
_CHUNK = 128
_GROUP = 32
_GROUPS_PER_CHUNK = _CHUNK // _GROUP
_N_ROWS = 1024 * 32 * 32
_N_CHUNKS = _N_ROWS // _CHUNK
_CHUNKS_PER_W = _N_CHUNKS // _NW
_UNROLL = 4
_NL_L = 32


def _lane_sum(v):
    lanes = lax.iota(jnp.int32, _L)
    dnums = lax.GatherDimensionNumbers(
        offset_dims=(), collapsed_slice_dims=(0,), start_index_map=(0,))
    for sh in (8, 4, 2, 1):
        perm = lax.gather(v, (lanes ^ sh)[:, None], dnums, slice_sizes=(1,),
                          mode=lax.GatherScatterMode.PROMISE_IN_BOUNDS)
        v = v + perm
    return v


def _rsqrt_nr(x):
    i = plsc.bitcast(x, jnp.int32)
    i = jnp.int32(0x5F3759DF) - (i >> 1)
    y = plsc.bitcast(i, jnp.float32)
    for _ in range(3):
        y = y * (jnp.float32(1.5) - jnp.float32(0.5) * x * y * y)
    return y


def _compute_chunk(ea, ob):
    for g in range(_GROUPS_PER_CHUNK):
        g0 = g * _GROUP

        def pass1(r, acc):
            s, ss = acc
            for u in range(_UNROLL):
                row = g0 + r * _UNROLL + u
                y0 = ea[row, pl.ds(0, _L)]
                y1 = ea[row, pl.ds(_L, _L)]
                s = s + (y0 + y1)
                ss = ss + (y0 * y0 + y1 * y1)
            return s, ss

        zero = jnp.zeros((_L,), jnp.float32)
        s, ss = lax.fori_loop(0, _GROUP // _UNROLL, pass1, (zero, zero))
        inv_n = jnp.float32(1.0 / (_GROUP * _EMBED))
        mean_v = _lane_sum(s) * inv_n
        var_v = jnp.maximum(_lane_sum(ss) * inv_n - mean_v * mean_v,
                            jnp.float32(0.0))
        scale_v = _rsqrt_nr(var_v + jnp.float32(1e-5))
        shift_v = mean_v * scale_v

        def pass2(r, carry):
            prow = (g0 >> 2) + r
            for u in range(_UNROLL):
                row = g0 + r * _UNROLL + u
                ob[prow, pl.ds(u * 32, _L)] = (
                    ea[row, pl.ds(0, _L)] * scale_v - shift_v)
                ob[prow, pl.ds(u * 32 + _L, _L)] = (
                    ea[row, pl.ds(_L, _L)] * scale_v - shift_v)
            return carry

        lax.fori_loop(0, _GROUP // _UNROLL, pass2, 0)


def _make_sc_kernel():
    mesh = plsc.VectorSubcoreMesh(core_axis_name="c", subcore_axis_name="s")
    f32 = jnp.float32

    @functools.partial(
        pl.kernel,
        mesh=mesh,
        compiler_params=pltpu.CompilerParams(needs_layout_passes=False,
                                             use_tc_tiling_on_sc=False),
        out_type=jax.ShapeDtypeStruct((_N_ROWS // 4, 4 * _EMBED), f32),
        scratch_types=[
            pltpu.VMEM((32, 4, 8, 32), f32),
            pltpu.VMEM((_CHUNKS_PER_W, _CHUNK), jnp.int32),
            pltpu.VMEM((_CHUNK, _EMBED), f32),
            pltpu.VMEM((_CHUNK // 4, 4 * _EMBED), f32),
            pltpu.VMEM((_CHUNK, _EMBED), f32),
            pltpu.VMEM((_CHUNK // 4, 4 * _EMBED), f32),
            pltpu.SemaphoreType.DMA,
            pltpu.SemaphoreType.DMA,
            pltpu.SemaphoreType.DMA,
            pltpu.SemaphoreType.DMA,
        ],
    )
    def sc_kernel(idx_hbm, tab_hbm, out_hbm, idx_n, idx_v,
                  ea0, ob0, ea1, ob1, sg0, sg1, so0, so1):
        wid = lax.axis_index("s") * _NC + lax.axis_index("c")
        chunk0 = wid * _CHUNKS_PER_W
        bufs = ((ea0, ob0, sg0, so0), (ea1, ob1, sg1, so1))

        pltpu.sync_copy(
            idx_hbm.at[:, :, wid >> 2, :, pl.ds((wid & 3) * 32, 32)], idx_n)
        iota16 = lax.iota(jnp.int32, _L)

        def stage(v, carry):
            il = jnp.full((_L,), (v >> 1) & 31, jnp.int32)
            id_ = iota16 + (v & 1) * _L
            ik = jnp.full((_L,), v >> 6, jnp.int32)
            vals = plsc.load_gather(idx_n, [il, id_ >> 3, id_ & 7, ik])
            idx_v[v >> 3, pl.ds((v & 7) * _L, _L)] = plsc.bitcast(
                vals, jnp.int32)
            return carry

        lax.fori_loop(0, 2048, stage, 0)

        for b in (0, 1):
            ea, _, sg, _ = bufs[b]
            pltpu.async_copy(tab_hbm.at[idx_v.at[b]], ea, sg)

        n_iter = _CHUNKS_PER_W // 2

        def body(j, carry):
            for b in (0, 1):
                ea, ob, sg, so = bufs[b]
                c = j * 2 + b
                pltpu.make_async_copy(tab_hbm.at[pl.ds(0, _CHUNK)], ea, sg).wait()

                @pl.when(j > 0)
                def _():
                    pltpu.make_async_copy(
                        ob, out_hbm.at[pl.ds(0, _CHUNK // 4)], so).wait()

                _compute_chunk(ea, ob)

                prow0 = (chunk0 + c) * (_CHUNK // 4)
                pltpu.async_copy(ob, out_hbm.at[pl.ds(prow0, _CHUNK // 4)], so)

                @pl.when(j < n_iter - 1)
                def _():
                    pltpu.async_copy(tab_hbm.at[idx_v.at[c + 2]], ea, sg)
            return carry

        lax.fori_loop(0, n_iter, body, 0)

        for b in (0, 1):
            _, ob, _, so = bufs[b]
            pltpu.make_async_copy(
                ob, out_hbm.at[pl.ds(0, _CHUNK // 4)], so).wait()

    return sc_kernel


_sc_kernel = _make_sc_kernel()


def kernel(x, emb_weight, pos_weight):
    b, l, d = x.shape
    e = emb_weight.shape[1]
    tab = emb_weight + pos_weight
    xt = (jnp.transpose(x, (1, 2, 0))
          .reshape(_NL_L, 4, 8, 8, 128)
          .transpose(0, 1, 3, 2, 4))
    xt = lax.bitcast_convert_type(xt, jnp.float32)
    out = _sc_kernel(xt, tab)
    return out.reshape(b, l, d, e)
